# baseline (speedup 1.0000x reference)
import functools

import jax
import jax.numpy as jnp
from jax import lax
from jax.experimental import pallas as pl
from jax.experimental.pallas import tpu as pltpu
from jax.experimental.pallas import tpu_sc as plsc

N = 10000
E = 160000
D = 268

NC = 2
NS = 16
NW = NC * NS

NP = 10240
SEG = NP // NS
FH = 134
FP = 144
DP = 384
EPW = 5120
EP = EPW * NW
K = 128
KB = 5
EPT = EP // NS
BN = 512
BN2 = 1024
_PAD_SPREAD = NP - N

_mesh = plsc.VectorSubcoreMesh(core_axis_name="c", subcore_axis_name="s")


@functools.partial(
    pl.kernel,
    out_type=jax.ShapeDtypeStruct((NC * NP,), jnp.float32),
    mesh=_mesh,
    scratch_types=[
        pltpu.VMEM((EPW,), jnp.int32),
        pltpu.VMEM((EPW,), jnp.float32),
        pltpu.VMEM_SHARED((NP,), jnp.float32),
    ],
)
def _deg_kernel(dstp, zeros1, onesw, out, idx_v, ones_v, acc):
    c = lax.axis_index("c")
    s = lax.axis_index("s")
    w = s * NC + c
    pltpu.sync_copy(zeros1.at[pl.ds(s * SEG, SEG)], acc.at[pl.ds(s * SEG, SEG)])
    pltpu.sync_copy(onesw, ones_v)
    pltpu.sync_copy(dstp.at[pl.ds(w * EPW, EPW)], idx_v)
    plsc.subcore_barrier()
    pltpu.sync_copy(ones_v, acc.at[idx_v], add=True)
    plsc.subcore_barrier()
    pltpu.sync_copy(acc.at[pl.ds(s * SEG, SEG)],
                    out.at[pl.ds(c * NP + s * SEG, SEG)])


@functools.partial(
    pl.kernel,
    out_type=jax.ShapeDtypeStruct((NC * NP, FP), jnp.float32),
    mesh=_mesh,
    compiler_params=pltpu.CompilerParams(use_tc_tiling_on_sc=False),
    scratch_types=[
        pltpu.VMEM((KB * K,), jnp.int32),
        pltpu.VMEM((KB * K,), jnp.int32),
        pltpu.VMEM((KB * K,), jnp.int32),
        pltpu.VMEM((K, FP), jnp.float32),
        pltpu.VMEM((K, FP), jnp.float32),
        pltpu.VMEM_SHARED((NP, FP), jnp.float32),
        pltpu.SemaphoreType.DMA,
        pltpu.SemaphoreType.DMA,
        pltpu.SemaphoreType.DMA,
        pltpu.SemaphoreType.DMA,
    ],
)
def _conv_kernel(xs2, src2, dstp, out, idx_sb, idx_db0, idx_db1,
                 rows0, rows1, acc, gsem0, gsem1, ssem0, ssem1):
    c = lax.axis_index("c")
    s = lax.axis_index("s")
    rows = (rows0, rows1)
    gsem = (gsem0, gsem1)
    ssem = (ssem0, ssem1)
    idx_db = (idx_db0, idx_db1)
    nch = EPT // K
    base = s * EPT

    pltpu.sync_copy(xs2.at[pl.ds(c * NP + s * SEG, SEG), :],
                    acc.at[pl.ds(s * SEG, SEG), :])
    plsc.subcore_barrier()

    pltpu.sync_copy(src2.at[pl.ds(c * EP + base, KB * K)], idx_sb)
    pltpu.sync_copy(dstp.at[pl.ds(base, KB * K)], idx_db[0])
    pltpu.async_copy(xs2.at[idx_sb.at[pl.ds(0, K)]], rows[0], gsem[0])

    def outer(go, carry):
        for j in range(2 * KB):
            g = go * (2 * KB) + j
            cur, nxt = j % 2, 1 - j % 2
            gn = g + 1
            dbuf = idx_db[(j // KB) % 2]

            pltpu.make_async_copy(
                xs2.at[idx_sb.at[pl.ds((j % KB) * K, K)]], rows[cur],
                gsem[cur]).wait()
            pltpu.async_copy(rows[cur],
                             acc.at[dbuf.at[pl.ds((j % KB) * K, K)]],
                             ssem[cur], add=True)

            def tail():
                @pl.when((go > 0) | (j > 0))
                def _():
                    pltpu.make_async_copy(
                        rows[nxt], acc.at[dbuf.at[pl.ds(0, K)]],
                        ssem[nxt]).wait()

                if (j + 1) % KB == 0:
                    nbuf = idx_db[((j // KB) + 1) % 2]
                    pltpu.sync_copy(
                        src2.at[pl.ds(c * EP + base + gn * K, KB * K)],
                        idx_sb)
                    pltpu.sync_copy(dstp.at[pl.ds(base + gn * K, KB * K)],
                                    nbuf)
                pltpu.async_copy(
                    xs2.at[idx_sb.at[pl.ds((gn % KB) * K, K)]],
                    rows[nxt], gsem[nxt])

            if j == 2 * KB - 1:
                @pl.when(gn < nch)
                def _():
                    tail()
            else:
                tail()
        return carry

    lax.fori_loop(0, nch // (2 * KB), outer, 0)
    pltpu.make_async_copy(rows[0], acc.at[idx_db[0].at[pl.ds(0, K)]],
                          ssem[0]).wait()
    pltpu.make_async_copy(rows[1], acc.at[idx_db[0].at[pl.ds(0, K)]],
                          ssem[1]).wait()
    plsc.subcore_barrier()
    pltpu.sync_copy(acc.at[pl.ds(s * SEG, SEG), :],
                    out.at[pl.ds(c * NP + s * SEG, SEG), :])


@functools.partial(
    pl.kernel,
    out_type=jax.ShapeDtypeStruct((NC * NP,), jnp.float32),
    mesh=_mesh,
    compiler_params=pltpu.CompilerParams(needs_layout_passes=False),
    scratch_types=[
        pltpu.VMEM((EPW,), jnp.int32),
        pltpu.VMEM((EPW,), jnp.int32),
        pltpu.VMEM((EPW,), jnp.float32),
        pltpu.VMEM((NP,), jnp.float32),
        pltpu.VMEM_SHARED((NP,), jnp.float32),
        pltpu.SemaphoreType.DMA,
    ],
)
def _sconv_kernel(sxs, src2, dstp, zeros1, out, idx_s, idx_d, vals, tab, acc,
                  sem):
    c = lax.axis_index("c")
    s = lax.axis_index("s")
    w = s * NC + c
    pltpu.sync_copy(zeros1.at[pl.ds(s * SEG, SEG)], acc.at[pl.ds(s * SEG, SEG)])
    pltpu.async_copy(sxs, tab, sem)
    pltpu.sync_copy(src2.at[pl.ds(w * EPW, EPW)], idx_s)
    pltpu.sync_copy(dstp.at[pl.ds(w * EPW, EPW)], idx_d)
    pltpu.make_async_copy(sxs, tab, sem).wait()

    def gat(j, carry):
        iv = idx_s[pl.ds(j * 16, 16)]
        vals[pl.ds(j * 16, 16)] = plsc.load_gather(tab, [iv])
        return carry

    lax.fori_loop(0, EPW // 16, gat, 0)
    plsc.subcore_barrier()
    pltpu.sync_copy(vals, acc.at[idx_d], add=True)
    plsc.subcore_barrier()
    pltpu.sync_copy(acc.at[pl.ds(s * SEG, SEG)],
                    out.at[pl.ds(c * NP + s * SEG, SEG)])


def _dinv_col(dg):
    tot = dg[0:1, :] + dg[1:2, :] + 1.0
    return jnp.transpose(lax.rsqrt(tot), (1, 0))


def _scale_body(x_ref, deg_ref, xs_ref):
    dv = _dinv_col(deg_ref[...])
    zpad = jnp.zeros((BN2, FP - FH), jnp.float32)
    xa = jnp.concatenate([x_ref[:, :FH], zpad], axis=1)
    xb = jnp.concatenate([x_ref[:, FH:], zpad[:, :FP - (D - FH)]], axis=1)
    xs_ref[...] = jnp.stack([xa, xb]) * dv[None, :, :]


def _scale_call(x, degf2):
    return pl.pallas_call(
        _scale_body,
        grid=(NP // BN2,),
        in_specs=[
            pl.BlockSpec((BN2, D), lambda n: (n, 0)),
            pl.BlockSpec((2, BN2), lambda n: (0, n)),
        ],
        out_specs=pl.BlockSpec((NC, BN2, FP), lambda n: (0, n, 0)),
        out_shape=jax.ShapeDtypeStruct((NC, NP, FP), jnp.float32),
    )(x, degf2)


def _mm_body(aga_ref, agb_ref, deg_ref, w1a_ref, w1b_ref, b1_ref, w2r_ref,
             g1_ref, bt1_ref, sxs_ref, h_s, st_s):
    p = pl.program_id(0)
    n = pl.program_id(1)
    dv = _dinv_col(deg_ref[...])
    rows = n * BN + lax.broadcasted_iota(jnp.int32, (BN, 1), 0)

    @pl.when(p == 0)
    def _():
        pa = aga_ref[0] * dv
        pb = agb_ref[0] * dv
        pre = (jnp.dot(pa, w1a_ref[...], preferred_element_type=jnp.float32)
               + jnp.dot(pb, w1b_ref[...], preferred_element_type=jnp.float32)
               + b1_ref[...])
        h = jnp.where(rows < N, jnp.maximum(pre, 0.0), 0.0)
        h_s[pl.ds(n * BN, BN), :] = h
        st = jnp.concatenate(
            [jnp.sum(h, axis=0, keepdims=True),
             jnp.sum(h * h, axis=0, keepdims=True)], axis=0)

        @pl.when(n == 0)
        def _():
            st_s[...] = st

        @pl.when(n > 0)
        def _():
            st_s[...] = st_s[...] + st

    @pl.when(p == 1)
    def _():
        st = st_s[...]
        mean = st[0:1, :] / float(N)
        var = st[1:2, :] / float(N) - mean * mean
        inv = lax.rsqrt(var + 1e-3)
        h = h_s[pl.ds(n * BN, BN), :]
        hb = jnp.maximum(g1_ref[...] * (h - mean) * inv + bt1_ref[...], 0.0)
        sv = jnp.sum(hb * w2r_ref[...], axis=1, keepdims=True)
        sv = jnp.where(rows < N, sv * dv, 0.0)
        sxs_ref[...] = jnp.transpose(sv, (1, 0))[None]


def _mm_call(agg, degf2, w1a, w1b, b1p, w2r, g1p, bt1p):
    return pl.pallas_call(
        _mm_body,
        grid=(2, NP // BN),
        in_specs=[
            pl.BlockSpec((1, BN, FP), lambda p, n: (0, n * (1 - p), 0)),
            pl.BlockSpec((1, BN, FP), lambda p, n: (1, n * (1 - p), 0)),
            pl.BlockSpec((2, BN), lambda p, n: (0, n)),
            pl.BlockSpec((FP, DP), lambda p, n: (0, 0)),
            pl.BlockSpec((FP, DP), lambda p, n: (0, 0)),
            pl.BlockSpec((1, DP), lambda p, n: (0, 0)),
            pl.BlockSpec((1, DP), lambda p, n: (0, 0)),
            pl.BlockSpec((1, DP), lambda p, n: (0, 0)),
            pl.BlockSpec((1, DP), lambda p, n: (0, 0)),
        ],
        out_specs=pl.BlockSpec((1, 1, BN), lambda p, n: (n, 0, 0)),
        out_shape=jax.ShapeDtypeStruct((NP // BN, 1, BN), jnp.float32),
        scratch_shapes=[
            pltpu.VMEM((NP, DP), jnp.float32),
            pltpu.VMEM((2, DP), jnp.float32),
        ],
    )(agg, agg, degf2, w1a, w1b, b1p, w2r, g1p, bt1p)


def _fin_body(sxs_ref, ag2_ref, deg_ref, b2_ref, g2_ref, bt2_ref, out_ref):
    dg = deg_ref[...]
    dinv = lax.rsqrt(dg[0] + dg[1] + 1.0)
    z = dinv * (sxs_ref[...] + ag2_ref[0] + ag2_ref[1]) + b2_ref[0, 0]
    r = lax.broadcasted_iota(jnp.int32, z.shape, 0)
    col = lax.broadcasted_iota(jnp.int32, z.shape, 1)
    valid = (r * 128 + col) < N
    zm = jnp.where(valid, z, 0.0)
    mean = jnp.sum(zm) / float(N)
    var = jnp.sum(jnp.where(valid, (z - mean) ** 2, 0.0)) / float(N)
    zn = g2_ref[0, 0] * (z - mean) * lax.rsqrt(var + 1e-3) + bt2_ref[0, 0]
    out_ref[...] = jax.nn.sigmoid(jnp.maximum(zn, 0.0))


def _fin_call(sxs2d, ag2, deg3, b2, g2, bt2):
    return pl.pallas_call(
        _fin_body,
        out_shape=jax.ShapeDtypeStruct((NP // 128, 128), jnp.float32),
    )(sxs2d, ag2, deg3, b2, g2, bt2)


def kernel(x, edge_index, edge_attr, W1, b1, gamma1, beta1, W2, b2, gamma2,
           beta2):
    f32 = jnp.float32
    src = edge_index[0]
    dst = edge_index[1]

    pad = N + (jnp.arange(EP - E, dtype=jnp.int32) % _PAD_SPREAD)
    srcp = jnp.concatenate([src, pad])
    dstp = jnp.concatenate([dst, pad])
    src2 = jnp.concatenate([srcp, srcp + NP])

    w1a = jnp.zeros((FP, DP), f32).at[:FH, :D].set(W1[:FH])
    w1b = jnp.zeros((FP, DP), f32).at[:D - FH, :D].set(W1[FH:])
    b1p = jnp.zeros((1, DP), f32).at[0, :D].set(b1)
    g1p = jnp.zeros((1, DP), f32).at[0, :D].set(gamma1)
    bt1p = jnp.zeros((1, DP), f32).at[0, :D].set(beta1)
    w2r = jnp.zeros((1, DP), f32).at[0, :D].set(W2[:, 0])

    zeros1 = jnp.zeros((NP,), f32)
    onesw = jnp.ones((EPW,), f32)

    degf = _deg_kernel(dstp, zeros1, onesw)
    degf2 = degf.reshape(NC, NP)

    xs2v = _scale_call(x, degf2)

    agg = _conv_kernel(xs2v.reshape(NC * NP, FP), src2, dstp)
    agg = agg.reshape(NC, NP, FP)

    sxs = _mm_call(agg, degf2, w1a, w1b, b1p, w2r, g1p, bt1p)

    agg2 = _sconv_kernel(sxs.reshape(NP), src2, dstp, zeros1)

    out2d = _fin_call(
        sxs.reshape(NP // 128, 128),
        agg2.reshape(NC, NP // 128, 128),
        degf.reshape(NC, NP // 128, 128),
        b2.reshape(1, 1), gamma2.reshape(1, 1), beta2.reshape(1, 1))
    return out2d.reshape(NP, 1)[:N]


# speedup vs baseline: 1.0425x; 1.0425x over previous
import functools

import jax
import jax.numpy as jnp
from jax import lax
from jax.experimental import pallas as pl
from jax.experimental.pallas import tpu as pltpu
from jax.experimental.pallas import tpu_sc as plsc

N = 10000
E = 160000
D = 268

NC = 2
NS = 16
NW = NC * NS

NP = 10240
SEG = NP // NS
FH = 134
FP = 144
DP = 384
EPW = 5120
EP = EPW * NW
K = 128
KB = 10
EPT = EP // NS
BN = 512
BN2 = 1024
_PAD_SPREAD = NP - N

_mesh = plsc.VectorSubcoreMesh(core_axis_name="c", subcore_axis_name="s")


@functools.partial(
    pl.kernel,
    out_type=jax.ShapeDtypeStruct((NC * NP,), jnp.float32),
    mesh=_mesh,
    scratch_types=[
        pltpu.VMEM((EPW,), jnp.int32),
        pltpu.VMEM((EPW,), jnp.float32),
        pltpu.VMEM_SHARED((NP,), jnp.float32),
    ],
)
def _deg_kernel(dstp, zeros1, onesw, out, idx_v, ones_v, acc):
    c = lax.axis_index("c")
    s = lax.axis_index("s")
    w = s * NC + c
    pltpu.sync_copy(zeros1.at[pl.ds(s * SEG, SEG)], acc.at[pl.ds(s * SEG, SEG)])
    pltpu.sync_copy(onesw, ones_v)
    pltpu.sync_copy(dstp.at[pl.ds(w * EPW, EPW)], idx_v)
    plsc.subcore_barrier()
    pltpu.sync_copy(ones_v, acc.at[idx_v], add=True)
    plsc.subcore_barrier()
    pltpu.sync_copy(acc.at[pl.ds(s * SEG, SEG)],
                    out.at[pl.ds(c * NP + s * SEG, SEG)])


@functools.partial(
    pl.kernel,
    out_type=jax.ShapeDtypeStruct((NC * NP, FP), jnp.float32),
    mesh=_mesh,
    compiler_params=pltpu.CompilerParams(use_tc_tiling_on_sc=False),
    scratch_types=[
        pltpu.VMEM((KB * K,), jnp.int32),
        pltpu.VMEM((K,), jnp.int32),
        pltpu.VMEM((K,), jnp.int32),
        pltpu.VMEM((K, FP), jnp.float32),
        pltpu.VMEM((K, FP), jnp.float32),
        pltpu.VMEM_SHARED((NP, FP), jnp.float32),
        pltpu.SemaphoreType.DMA,
        pltpu.SemaphoreType.DMA,
        pltpu.SemaphoreType.DMA,
        pltpu.SemaphoreType.DMA,
    ],
)
def _conv_kernel(xs2, src2, dstp, out, idx_sb, idx_d0, idx_d1,
                 rows0, rows1, acc, gsem0, gsem1, ssem0, ssem1):
    c = lax.axis_index("c")
    s = lax.axis_index("s")
    idx_d = (idx_d0, idx_d1)
    rows = (rows0, rows1)
    gsem = (gsem0, gsem1)
    ssem = (ssem0, ssem1)
    nch = EPT // K
    base = s * EPT

    pltpu.sync_copy(xs2.at[pl.ds(c * NP + s * SEG, SEG), :],
                    acc.at[pl.ds(s * SEG, SEG), :])
    plsc.subcore_barrier()

    pltpu.sync_copy(src2.at[pl.ds(c * EP + base, KB * K)], idx_sb)
    pltpu.async_copy(xs2.at[idx_sb.at[pl.ds(0, K)]], rows[0], gsem[0])

    def outer(go, carry):
        for b in (0, 1):
            g = go * 2 + b
            cur, nxt = b, 1 - b
            gn = g + 1

            pltpu.sync_copy(dstp.at[pl.ds(base + g * K, K)], idx_d[cur])
            pltpu.make_async_copy(
                xs2.at[idx_sb.at[pl.ds((g % KB) * K, K)]], rows[cur],
                gsem[cur]).wait()

            @pl.when(gn < nch)
            def _():
                @pl.when(gn % KB == 0)
                def _():
                    pltpu.sync_copy(
                        src2.at[pl.ds(c * EP + base + gn * K, KB * K)],
                        idx_sb)

                @pl.when(g >= 1)
                def _():
                    pltpu.make_async_copy(rows[nxt], acc.at[idx_d[nxt]],
                                          ssem[nxt]).wait()

                pltpu.async_copy(xs2.at[idx_sb.at[pl.ds((gn % KB) * K, K)]],
                                 rows[nxt], gsem[nxt])

            pltpu.async_copy(rows[cur], acc.at[idx_d[cur]], ssem[cur],
                             add=True)
        return carry

    lax.fori_loop(0, nch // 2, outer, 0)
    pltpu.make_async_copy(rows[0], acc.at[idx_d[0]], ssem[0]).wait()
    pltpu.make_async_copy(rows[1], acc.at[idx_d[1]], ssem[1]).wait()
    plsc.subcore_barrier()
    pltpu.sync_copy(acc.at[pl.ds(s * SEG, SEG), :],
                    out.at[pl.ds(c * NP + s * SEG, SEG), :])


@functools.partial(
    pl.kernel,
    out_type=jax.ShapeDtypeStruct((NC * NP,), jnp.float32),
    mesh=_mesh,
    compiler_params=pltpu.CompilerParams(needs_layout_passes=False),
    scratch_types=[
        pltpu.VMEM((EPW,), jnp.int32),
        pltpu.VMEM((EPW,), jnp.int32),
        pltpu.VMEM((EPW,), jnp.float32),
        pltpu.VMEM((NP,), jnp.float32),
        pltpu.VMEM_SHARED((NP,), jnp.float32),
        pltpu.SemaphoreType.DMA,
    ],
)
def _sconv_kernel(sxs, src2, dstp, zeros1, out, idx_s, idx_d, vals, tab, acc,
                  sem):
    c = lax.axis_index("c")
    s = lax.axis_index("s")
    w = s * NC + c
    pltpu.sync_copy(zeros1.at[pl.ds(s * SEG, SEG)], acc.at[pl.ds(s * SEG, SEG)])
    pltpu.async_copy(sxs, tab, sem)
    pltpu.sync_copy(src2.at[pl.ds(w * EPW, EPW)], idx_s)
    pltpu.sync_copy(dstp.at[pl.ds(w * EPW, EPW)], idx_d)
    pltpu.make_async_copy(sxs, tab, sem).wait()

    def gat(j, carry):
        iv = idx_s[pl.ds(j * 16, 16)]
        vals[pl.ds(j * 16, 16)] = plsc.load_gather(tab, [iv])
        return carry

    lax.fori_loop(0, EPW // 16, gat, 0)
    plsc.subcore_barrier()
    pltpu.sync_copy(vals, acc.at[idx_d], add=True)
    plsc.subcore_barrier()
    pltpu.sync_copy(acc.at[pl.ds(s * SEG, SEG)],
                    out.at[pl.ds(c * NP + s * SEG, SEG)])


def _dinv_col(dg):
    tot = dg[0:1, :] + dg[1:2, :] + 1.0
    return jnp.transpose(lax.rsqrt(tot), (1, 0))


def _scale_body(x_ref, deg_ref, xs_ref):
    dv = _dinv_col(deg_ref[...])
    zpad = jnp.zeros((BN2, FP - FH), jnp.float32)
    xa = jnp.concatenate([x_ref[:, :FH], zpad], axis=1)
    xb = jnp.concatenate([x_ref[:, FH:], zpad[:, :FP - (D - FH)]], axis=1)
    xs_ref[...] = jnp.stack([xa, xb]) * dv[None, :, :]


def _scale_call(x, degf2):
    return pl.pallas_call(
        _scale_body,
        grid=(NP // BN2,),
        in_specs=[
            pl.BlockSpec((BN2, D), lambda n: (n, 0)),
            pl.BlockSpec((2, BN2), lambda n: (0, n)),
        ],
        out_specs=pl.BlockSpec((NC, BN2, FP), lambda n: (0, n, 0)),
        out_shape=jax.ShapeDtypeStruct((NC, NP, FP), jnp.float32),
    )(x, degf2)


def _mm_body(aga_ref, agb_ref, deg_ref, w1a_ref, w1b_ref, b1_ref, w2r_ref,
             g1_ref, bt1_ref, sxs_ref, h_s, st_s):
    p = pl.program_id(0)
    n = pl.program_id(1)
    dv = _dinv_col(deg_ref[...])
    rows = n * BN + lax.broadcasted_iota(jnp.int32, (BN, 1), 0)

    @pl.when(p == 0)
    def _():
        pa = aga_ref[0] * dv
        pb = agb_ref[0] * dv
        pre = (jnp.dot(pa, w1a_ref[...], preferred_element_type=jnp.float32)
               + jnp.dot(pb, w1b_ref[...], preferred_element_type=jnp.float32)
               + b1_ref[...])
        h = jnp.where(rows < N, jnp.maximum(pre, 0.0), 0.0)
        h_s[pl.ds(n * BN, BN), :] = h
        st = jnp.concatenate(
            [jnp.sum(h, axis=0, keepdims=True),
             jnp.sum(h * h, axis=0, keepdims=True)], axis=0)

        @pl.when(n == 0)
        def _():
            st_s[...] = st

        @pl.when(n > 0)
        def _():
            st_s[...] = st_s[...] + st

    @pl.when(p == 1)
    def _():
        st = st_s[...]
        mean = st[0:1, :] / float(N)
        var = st[1:2, :] / float(N) - mean * mean
        inv = lax.rsqrt(var + 1e-3)
        h = h_s[pl.ds(n * BN, BN), :]
        hb = jnp.maximum(g1_ref[...] * (h - mean) * inv + bt1_ref[...], 0.0)
        sv = jnp.sum(hb * w2r_ref[...], axis=1, keepdims=True)
        sv = jnp.where(rows < N, sv * dv, 0.0)
        sxs_ref[...] = jnp.transpose(sv, (1, 0))[None]


def _mm_call(agg, degf2, w1a, w1b, b1p, w2r, g1p, bt1p):
    return pl.pallas_call(
        _mm_body,
        grid=(2, NP // BN),
        in_specs=[
            pl.BlockSpec((1, BN, FP), lambda p, n: (0, n * (1 - p), 0)),
            pl.BlockSpec((1, BN, FP), lambda p, n: (1, n * (1 - p), 0)),
            pl.BlockSpec((2, BN), lambda p, n: (0, n)),
            pl.BlockSpec((FP, DP), lambda p, n: (0, 0)),
            pl.BlockSpec((FP, DP), lambda p, n: (0, 0)),
            pl.BlockSpec((1, DP), lambda p, n: (0, 0)),
            pl.BlockSpec((1, DP), lambda p, n: (0, 0)),
            pl.BlockSpec((1, DP), lambda p, n: (0, 0)),
            pl.BlockSpec((1, DP), lambda p, n: (0, 0)),
        ],
        out_specs=pl.BlockSpec((1, 1, BN), lambda p, n: (n, 0, 0)),
        out_shape=jax.ShapeDtypeStruct((NP // BN, 1, BN), jnp.float32),
        scratch_shapes=[
            pltpu.VMEM((NP, DP), jnp.float32),
            pltpu.VMEM((2, DP), jnp.float32),
        ],
    )(agg, agg, degf2, w1a, w1b, b1p, w2r, g1p, bt1p)


def _fin_body(sxs_ref, ag2_ref, deg_ref, b2_ref, g2_ref, bt2_ref, out_ref):
    dg = deg_ref[...]
    dinv = lax.rsqrt(dg[0] + dg[1] + 1.0)
    z = dinv * (sxs_ref[...] + ag2_ref[0] + ag2_ref[1]) + b2_ref[0, 0]
    r = lax.broadcasted_iota(jnp.int32, z.shape, 0)
    col = lax.broadcasted_iota(jnp.int32, z.shape, 1)
    valid = (r * 128 + col) < N
    zm = jnp.where(valid, z, 0.0)
    mean = jnp.sum(zm) / float(N)
    var = jnp.sum(jnp.where(valid, (z - mean) ** 2, 0.0)) / float(N)
    zn = g2_ref[0, 0] * (z - mean) * lax.rsqrt(var + 1e-3) + bt2_ref[0, 0]
    out_ref[...] = jax.nn.sigmoid(jnp.maximum(zn, 0.0))


def _fin_call(sxs2d, ag2, deg3, b2, g2, bt2):
    return pl.pallas_call(
        _fin_body,
        out_shape=jax.ShapeDtypeStruct((NP // 128, 128), jnp.float32),
    )(sxs2d, ag2, deg3, b2, g2, bt2)


def kernel(x, edge_index, edge_attr, W1, b1, gamma1, beta1, W2, b2, gamma2,
           beta2):
    f32 = jnp.float32
    src = edge_index[0]
    dst = edge_index[1]

    pad = N + (jnp.arange(EP - E, dtype=jnp.int32) % _PAD_SPREAD)
    srcp = jnp.concatenate([src, pad])
    dstp = jnp.concatenate([dst, pad])
    src2 = jnp.concatenate([srcp, srcp + NP])

    w1a = jnp.zeros((FP, DP), f32).at[:FH, :D].set(W1[:FH])
    w1b = jnp.zeros((FP, DP), f32).at[:D - FH, :D].set(W1[FH:])
    b1p = jnp.zeros((1, DP), f32).at[0, :D].set(b1)
    g1p = jnp.zeros((1, DP), f32).at[0, :D].set(gamma1)
    bt1p = jnp.zeros((1, DP), f32).at[0, :D].set(beta1)
    w2r = jnp.zeros((1, DP), f32).at[0, :D].set(W2[:, 0])

    zeros1 = jnp.zeros((NP,), f32)
    onesw = jnp.ones((EPW,), f32)

    degf = _deg_kernel(dstp, zeros1, onesw)
    degf2 = degf.reshape(NC, NP)

    xs2v = _scale_call(x, degf2)

    agg = _conv_kernel(xs2v.reshape(NC * NP, FP), src2, dstp)
    agg = agg.reshape(NC, NP, FP)

    sxs = _mm_call(agg, degf2, w1a, w1b, b1p, w2r, g1p, bt1p)

    agg2 = _sconv_kernel(sxs.reshape(NP), src2, dstp, zeros1)

    out2d = _fin_call(
        sxs.reshape(NP // 128, 128),
        agg2.reshape(NC, NP // 128, 128),
        degf.reshape(NC, NP // 128, 128),
        b2.reshape(1, 1), gamma2.reshape(1, 1), beta2.reshape(1, 1))
    return out2d.reshape(NP, 1)[:N]


# speedup vs baseline: 1.1531x; 1.1062x over previous
import functools

import jax
import jax.numpy as jnp
from jax import lax
from jax.experimental import pallas as pl
from jax.experimental.pallas import tpu as pltpu
from jax.experimental.pallas import tpu_sc as plsc

N = 10000
E = 160000
D = 268

NC = 2
NS = 16
NW = NC * NS

NP = 10240
SEG = NP // NS
FH = 134
FP = 160
DP = 384
EPW = 5120
EP = EPW * NW
K = 256
KB = 10
EPT = EP // NS
BN = 512
BN2 = 1024
_PAD_SPREAD = NP - N

_mesh = plsc.VectorSubcoreMesh(core_axis_name="c", subcore_axis_name="s")


@functools.partial(
    pl.kernel,
    out_type=jax.ShapeDtypeStruct((NC * NP,), jnp.float32),
    mesh=_mesh,
    scratch_types=[
        pltpu.VMEM((EPW,), jnp.int32),
        pltpu.VMEM((EPW,), jnp.float32),
        pltpu.VMEM_SHARED((NP,), jnp.float32),
    ],
)
def _deg_kernel(dstp, zeros1, onesw, out, idx_v, ones_v, acc):
    c = lax.axis_index("c")
    s = lax.axis_index("s")
    w = s * NC + c
    pltpu.sync_copy(zeros1.at[pl.ds(s * SEG, SEG)], acc.at[pl.ds(s * SEG, SEG)])
    pltpu.sync_copy(onesw, ones_v)
    pltpu.sync_copy(dstp.at[pl.ds(w * EPW, EPW)], idx_v)
    plsc.subcore_barrier()
    pltpu.sync_copy(ones_v, acc.at[idx_v], add=True)
    plsc.subcore_barrier()
    pltpu.sync_copy(acc.at[pl.ds(s * SEG, SEG)],
                    out.at[pl.ds(c * NP + s * SEG, SEG)])


@functools.partial(
    pl.kernel,
    out_type=jax.ShapeDtypeStruct((NC * NP, FP), jnp.bfloat16),
    mesh=_mesh,
    compiler_params=pltpu.CompilerParams(use_tc_tiling_on_sc=False),
    scratch_types=[
        pltpu.VMEM((KB * K,), jnp.int32),
        pltpu.VMEM((K,), jnp.int32),
        pltpu.VMEM((K,), jnp.int32),
        pltpu.VMEM((K, FP), jnp.bfloat16),
        pltpu.VMEM((K, FP), jnp.bfloat16),
        pltpu.VMEM_SHARED((NP, FP), jnp.bfloat16),
        pltpu.SemaphoreType.DMA,
        pltpu.SemaphoreType.DMA,
        pltpu.SemaphoreType.DMA,
        pltpu.SemaphoreType.DMA,
    ],
)
def _conv_kernel(xs2, src2, dstp, out, idx_sb, idx_d0, idx_d1,
                 rows0, rows1, acc, gsem0, gsem1, ssem0, ssem1):
    c = lax.axis_index("c")
    s = lax.axis_index("s")
    idx_d = (idx_d0, idx_d1)
    rows = (rows0, rows1)
    gsem = (gsem0, gsem1)
    ssem = (ssem0, ssem1)
    nch = EPT // K
    base = s * EPT

    pltpu.sync_copy(xs2.at[pl.ds(c * NP + s * SEG, SEG), :],
                    acc.at[pl.ds(s * SEG, SEG), :])
    plsc.subcore_barrier()

    pltpu.sync_copy(src2.at[pl.ds(c * EP + base, KB * K)], idx_sb)
    pltpu.async_copy(xs2.at[idx_sb.at[pl.ds(0, K)]], rows[0], gsem[0])

    def outer(go, carry):
        for b in (0, 1):
            g = go * 2 + b
            cur, nxt = b, 1 - b
            gn = g + 1

            pltpu.sync_copy(dstp.at[pl.ds(base + g * K, K)], idx_d[cur])
            pltpu.make_async_copy(
                xs2.at[idx_sb.at[pl.ds((g % KB) * K, K)]], rows[cur],
                gsem[cur]).wait()

            @pl.when(gn < nch)
            def _():
                @pl.when(gn % KB == 0)
                def _():
                    pltpu.sync_copy(
                        src2.at[pl.ds(c * EP + base + gn * K, KB * K)],
                        idx_sb)

                @pl.when(g >= 1)
                def _():
                    pltpu.make_async_copy(rows[nxt], acc.at[idx_d[nxt]],
                                          ssem[nxt]).wait()

                pltpu.async_copy(xs2.at[idx_sb.at[pl.ds((gn % KB) * K, K)]],
                                 rows[nxt], gsem[nxt])

            pltpu.async_copy(rows[cur], acc.at[idx_d[cur]], ssem[cur],
                             add=True)
        return carry

    lax.fori_loop(0, nch // 2, outer, 0)
    pltpu.make_async_copy(rows[0], acc.at[idx_d[0]], ssem[0]).wait()
    pltpu.make_async_copy(rows[1], acc.at[idx_d[1]], ssem[1]).wait()
    plsc.subcore_barrier()
    pltpu.sync_copy(acc.at[pl.ds(s * SEG, SEG), :],
                    out.at[pl.ds(c * NP + s * SEG, SEG), :])


@functools.partial(
    pl.kernel,
    out_type=jax.ShapeDtypeStruct((NC * NP,), jnp.float32),
    mesh=_mesh,
    compiler_params=pltpu.CompilerParams(needs_layout_passes=False),
    scratch_types=[
        pltpu.VMEM((EPW,), jnp.int32),
        pltpu.VMEM((EPW,), jnp.int32),
        pltpu.VMEM((EPW,), jnp.float32),
        pltpu.VMEM((NP,), jnp.float32),
        pltpu.VMEM_SHARED((NP,), jnp.float32),
        pltpu.SemaphoreType.DMA,
    ],
)
def _sconv_kernel(sxs, src2, dstp, zeros1, out, idx_s, idx_d, vals, tab, acc,
                  sem):
    c = lax.axis_index("c")
    s = lax.axis_index("s")
    w = s * NC + c
    pltpu.sync_copy(zeros1.at[pl.ds(s * SEG, SEG)], acc.at[pl.ds(s * SEG, SEG)])
    pltpu.async_copy(sxs, tab, sem)
    pltpu.sync_copy(src2.at[pl.ds(w * EPW, EPW)], idx_s)
    pltpu.sync_copy(dstp.at[pl.ds(w * EPW, EPW)], idx_d)
    pltpu.make_async_copy(sxs, tab, sem).wait()

    def gat(j, carry):
        iv = idx_s[pl.ds(j * 16, 16)]
        vals[pl.ds(j * 16, 16)] = plsc.load_gather(tab, [iv])
        return carry

    lax.fori_loop(0, EPW // 16, gat, 0)
    plsc.subcore_barrier()
    pltpu.sync_copy(vals, acc.at[idx_d], add=True)
    plsc.subcore_barrier()
    pltpu.sync_copy(acc.at[pl.ds(s * SEG, SEG)],
                    out.at[pl.ds(c * NP + s * SEG, SEG)])


def _dinv_col(dg):
    tot = dg[0:1, :] + dg[1:2, :] + 1.0
    return jnp.transpose(lax.rsqrt(tot), (1, 0))


def _scale_body(x_ref, deg_ref, xs_ref):
    dv = _dinv_col(deg_ref[...])
    zpad = jnp.zeros((BN2, FP - FH), jnp.float32)
    xa = jnp.concatenate([x_ref[:, :FH], zpad], axis=1)
    xb = jnp.concatenate([x_ref[:, FH:], zpad[:, :FP - (D - FH)]], axis=1)
    xs_ref[...] = (jnp.stack([xa, xb]) * dv[None, :, :]).astype(jnp.bfloat16)


def _scale_call(x, degf2):
    return pl.pallas_call(
        _scale_body,
        grid=(NP // BN2,),
        in_specs=[
            pl.BlockSpec((BN2, D), lambda n: (n, 0)),
            pl.BlockSpec((2, BN2), lambda n: (0, n)),
        ],
        out_specs=pl.BlockSpec((NC, BN2, FP), lambda n: (0, n, 0)),
        out_shape=jax.ShapeDtypeStruct((NC, NP, FP), jnp.bfloat16),
    )(x, degf2)


def _mm_body(aga_ref, agb_ref, deg_ref, w1a_ref, w1b_ref, b1_ref, w2r_ref,
             g1_ref, bt1_ref, sxs_ref, h_s, st_s):
    p = pl.program_id(0)
    n = pl.program_id(1)
    dv = _dinv_col(deg_ref[...])
    rows = n * BN + lax.broadcasted_iota(jnp.int32, (BN, 1), 0)

    @pl.when(p == 0)
    def _():
        pa = aga_ref[0].astype(jnp.float32) * dv
        pb = agb_ref[0].astype(jnp.float32) * dv
        pre = (jnp.dot(pa, w1a_ref[...], preferred_element_type=jnp.float32)
               + jnp.dot(pb, w1b_ref[...], preferred_element_type=jnp.float32)
               + b1_ref[...])
        h = jnp.where(rows < N, jnp.maximum(pre, 0.0), 0.0)
        h_s[pl.ds(n * BN, BN), :] = h
        st = jnp.concatenate(
            [jnp.sum(h, axis=0, keepdims=True),
             jnp.sum(h * h, axis=0, keepdims=True)], axis=0)

        @pl.when(n == 0)
        def _():
            st_s[...] = st

        @pl.when(n > 0)
        def _():
            st_s[...] = st_s[...] + st

    @pl.when(p == 1)
    def _():
        st = st_s[...]
        mean = st[0:1, :] / float(N)
        var = st[1:2, :] / float(N) - mean * mean
        inv = lax.rsqrt(var + 1e-3)
        h = h_s[pl.ds(n * BN, BN), :]
        hb = jnp.maximum(g1_ref[...] * (h - mean) * inv + bt1_ref[...], 0.0)
        sv = jnp.sum(hb * w2r_ref[...], axis=1, keepdims=True)
        sv = jnp.where(rows < N, sv * dv, 0.0)
        sxs_ref[...] = jnp.transpose(sv, (1, 0))[None]


def _mm_call(agg, degf2, w1a, w1b, b1p, w2r, g1p, bt1p):
    return pl.pallas_call(
        _mm_body,
        grid=(2, NP // BN),
        in_specs=[
            pl.BlockSpec((1, BN, FP), lambda p, n: (0, n * (1 - p), 0)),
            pl.BlockSpec((1, BN, FP), lambda p, n: (1, n * (1 - p), 0)),
            pl.BlockSpec((2, BN), lambda p, n: (0, n)),
            pl.BlockSpec((FP, DP), lambda p, n: (0, 0)),
            pl.BlockSpec((FP, DP), lambda p, n: (0, 0)),
            pl.BlockSpec((1, DP), lambda p, n: (0, 0)),
            pl.BlockSpec((1, DP), lambda p, n: (0, 0)),
            pl.BlockSpec((1, DP), lambda p, n: (0, 0)),
            pl.BlockSpec((1, DP), lambda p, n: (0, 0)),
        ],
        out_specs=pl.BlockSpec((1, 1, BN), lambda p, n: (n, 0, 0)),
        out_shape=jax.ShapeDtypeStruct((NP // BN, 1, BN), jnp.float32),
        scratch_shapes=[
            pltpu.VMEM((NP, DP), jnp.float32),
            pltpu.VMEM((2, DP), jnp.float32),
        ],
    )(agg, agg, degf2, w1a, w1b, b1p, w2r, g1p, bt1p)


def _fin_body(sxs_ref, ag2_ref, deg_ref, b2_ref, g2_ref, bt2_ref, out_ref):
    dg = deg_ref[...]
    dinv = lax.rsqrt(dg[0] + dg[1] + 1.0)
    z = dinv * (sxs_ref[...] + ag2_ref[0] + ag2_ref[1]) + b2_ref[0, 0]
    r = lax.broadcasted_iota(jnp.int32, z.shape, 0)
    col = lax.broadcasted_iota(jnp.int32, z.shape, 1)
    valid = (r * 128 + col) < N
    zm = jnp.where(valid, z, 0.0)
    mean = jnp.sum(zm) / float(N)
    var = jnp.sum(jnp.where(valid, (z - mean) ** 2, 0.0)) / float(N)
    zn = g2_ref[0, 0] * (z - mean) * lax.rsqrt(var + 1e-3) + bt2_ref[0, 0]
    out_ref[...] = jax.nn.sigmoid(jnp.maximum(zn, 0.0))


def _fin_call(sxs2d, ag2, deg3, b2, g2, bt2):
    return pl.pallas_call(
        _fin_body,
        out_shape=jax.ShapeDtypeStruct((NP // 128, 128), jnp.float32),
    )(sxs2d, ag2, deg3, b2, g2, bt2)


def kernel(x, edge_index, edge_attr, W1, b1, gamma1, beta1, W2, b2, gamma2,
           beta2):
    f32 = jnp.float32
    src = edge_index[0]
    dst = edge_index[1]

    pad = N + (jnp.arange(EP - E, dtype=jnp.int32) % _PAD_SPREAD)
    srcp = jnp.concatenate([src, pad])
    dstp = jnp.concatenate([dst, pad])
    src2 = jnp.concatenate([srcp, srcp + NP])

    w1a = jnp.zeros((FP, DP), f32).at[:FH, :D].set(W1[:FH])
    w1b = jnp.zeros((FP, DP), f32).at[:D - FH, :D].set(W1[FH:])
    b1p = jnp.zeros((1, DP), f32).at[0, :D].set(b1)
    g1p = jnp.zeros((1, DP), f32).at[0, :D].set(gamma1)
    bt1p = jnp.zeros((1, DP), f32).at[0, :D].set(beta1)
    w2r = jnp.zeros((1, DP), f32).at[0, :D].set(W2[:, 0])

    zeros1 = jnp.zeros((NP,), f32)
    onesw = jnp.ones((EPW,), f32)

    degf = _deg_kernel(dstp, zeros1, onesw)
    degf2 = degf.reshape(NC, NP)

    xs2v = _scale_call(x, degf2)

    agg = _conv_kernel(xs2v.reshape(NC * NP, FP), src2, dstp)
    agg = agg.reshape(NC, NP, FP)

    sxs = _mm_call(agg, degf2, w1a, w1b, b1p, w2r, g1p, bt1p)

    agg2 = _sconv_kernel(sxs.reshape(NP), src2, dstp, zeros1)

    out2d = _fin_call(
        sxs.reshape(NP // 128, 128),
        agg2.reshape(NC, NP // 128, 128),
        degf.reshape(NC, NP // 128, 128),
        b2.reshape(1, 1), gamma2.reshape(1, 1), beta2.reshape(1, 1))
    return out2d.reshape(NP, 1)[:N]
